# Initial kernel scaffold; baseline (speedup 1.0000x reference)
import jax
import jax.numpy as jnp
from jax.experimental import pallas as pl


def kernel(node_subgraph, adj_row, adj_col, adj_val, feat_full, label_full, W0_0, b0_0, W0_1, b0_1, W1_0, b1_0, W1_1, b1_1, Wc, bc):
    raise NotImplementedError("write your pallas kernel here")


# speedup vs baseline: 3.6476x; 3.6476x over previous
import functools

import jax
import jax.numpy as jnp
from jax import lax
from jax.experimental import pallas as pl
from jax.experimental.pallas import tpu as pltpu
from jax.experimental.pallas import tpu_sc as plsc

_F32 = jnp.float32
_I32 = jnp.int32


def _sc_gather(feat_full, lab_idx_full, idx_pad):
    n_full, d = feat_full.shape
    b = idx_pad.shape[0]
    bpw = b // 32

    mesh = plsc.VectorSubcoreMesh(core_axis_name="c", subcore_axis_name="s")

    @functools.partial(
        pl.kernel,
        out_type=(
            jax.ShapeDtypeStruct((b, d), _F32),
            jax.ShapeDtypeStruct((b,), _I32),
        ),
        mesh=mesh,
        compiler_params=pltpu.CompilerParams(needs_layout_passes=False),
        scratch_types=[
            pltpu.VMEM((bpw // 2,), _I32),
            pltpu.VMEM((bpw // 2,), _I32),
            pltpu.VMEM((bpw // 2, d), _F32),
            pltpu.VMEM((n_full,), _I32),
            pltpu.VMEM((bpw,), _I32),
            pltpu.SemaphoreType.DMA,
            pltpu.SemaphoreType.DMA,
        ],
    )
    def k(feat_hbm, lab_hbm, idx_hbm, feat_out, lab_out,
          idx_a, idx_b, frows, lab_v, lout_v, sem, sem2):
        c = lax.axis_index("c")
        s = lax.axis_index("s")
        wid = s * 2 + c
        base = wid * bpw
        half = bpw // 2
        pltpu.sync_copy(idx_hbm.at[pl.ds(base, half)], idx_a)
        pltpu.sync_copy(idx_hbm.at[pl.ds(base + half, half)], idx_b)
        cp = pltpu.make_async_copy(lab_hbm, lab_v, sem2)
        cp.start()
        pltpu.async_copy(feat_hbm.at[idx_a], frows, sem).wait()
        pltpu.sync_copy(frows, feat_out.at[pl.ds(base, half)])
        pltpu.async_copy(feat_hbm.at[idx_b], frows, sem).wait()
        pltpu.sync_copy(frows, feat_out.at[pl.ds(base + half, half)])
        cp.wait()
        for j in range(half // 16):
            iv = idx_a[pl.ds(j * 16, 16)]
            lout_v[pl.ds(j * 16, 16)] = plsc.load_gather(lab_v, [iv])
        for j in range(half // 16):
            iv = idx_b[pl.ds(j * 16, 16)]
            lout_v[pl.ds(half + j * 16, 16)] = plsc.load_gather(lab_v, [iv])
        pltpu.sync_copy(lout_v, lab_out.at[pl.ds(base, bpw)])

    return k(feat_full, lab_idx_full, idx_pad)


def _sc_spmm(row2d, col2d, val2d, h):
    n, d = h.shape
    n_pad = ((n + 511) // 512) * 512
    nch = row2d.shape[0]
    nch_t = nch // 32
    k_ch = row2d.shape[1]
    ngrp = 4
    gch = nch_t // ngrp
    rpt = n_pad // 16
    ng = d // 16

    mesh = plsc.VectorSubcoreMesh(core_axis_name="c", subcore_axis_name="s")

    @functools.partial(
        pl.kernel,
        out_type=jax.ShapeDtypeStruct((2 * n_pad, d), _F32),
        mesh=mesh,
        compiler_params=pltpu.CompilerParams(needs_layout_passes=False),
        scratch_types=[
            [pltpu.VMEM((gch, k_ch), _I32)] * 2,
            [pltpu.VMEM((gch, k_ch), _I32)] * 2,
            [pltpu.VMEM((gch, k_ch), _F32)] * 2,
            [pltpu.VMEM((k_ch, d), _F32)] * 2,
            pltpu.MemorySpace.VMEM_SHARED((n_pad, d), _F32),
            pltpu.SemaphoreType.DMA,
            [pltpu.SemaphoreType.DMA] * 2,
        ],
    )
    def k(row_hbm, col_hbm, val_hbm, h_hbm, out_hbm,
          col_g, row_g, val_g, rows, acc, sem_i, gsem):
        c = lax.axis_index("c")
        s = lax.axis_index("s")
        wid = s * 2 + c
        cb = wid * nch_t

        def load_group(grp):
            p = grp % 2
            off = cb + grp * gch
            pltpu.async_copy(col_hbm.at[pl.ds(off, gch)], col_g[p], sem_i)
            pltpu.async_copy(row_hbm.at[pl.ds(off, gch)], row_g[p], sem_i)
            pltpu.async_copy(val_hbm.at[pl.ds(off, gch)], val_g[p], sem_i)

        def wait_group(grp):
            p = grp % 2
            pltpu.make_async_copy(col_hbm.at[pl.ds(0, gch)], col_g[p], sem_i).wait()
            pltpu.make_async_copy(row_hbm.at[pl.ds(0, gch)], row_g[p], sem_i).wait()
            pltpu.make_async_copy(val_hbm.at[pl.ds(0, gch)], val_g[p], sem_i).wait()

        def gather(p, lc, b):
            pltpu.async_copy(h_hbm.at[col_g[p].at[lc]], rows[b], gsem[b])

        def wait_gather(b):
            pltpu.make_async_copy(
                h_hbm.at[col_g[0].at[0]], rows[b], gsem[b]).wait()

        def scale(p, lc, b):
            def sj(j):
                vv = val_g[p][lc, pl.ds(j * 16, 16)]
                for t in range(16):
                    for g in range(ng):
                        sl = pl.ds(g * 16, 16)
                        rows[b][j * 16 + t, sl] = rows[b][j * 16 + t, sl] * vv[t]
            plsc.parallel_loop(0, k_ch // 16)(sj)

        def scatter(p, lc, b):
            pltpu.sync_copy(rows[b], acc.at[row_g[p].at[lc]], add=True)

        load_group(0)

        zero16 = jnp.zeros((16,), _F32)

        def zfill(i, carry):
            for t in range(16):
                for g in range(ng):
                    rows[0][i * 16 + t, pl.ds(g * 16, 16)] = zero16
            return carry

        lax.fori_loop(0, k_ch // 16, zfill, 0)
        r0 = s * rpt
        for j in range(rpt // k_ch):
            pltpu.sync_copy(rows[0], acc.at[pl.ds(r0 + j * k_ch, k_ch)])
        wait_group(0)
        plsc.subcore_barrier()

        gather(0, 0, 0)
        gather(0, 1, 1)

        for grp in range(ngrp):
            p = grp % 2
            if grp + 1 < ngrp:
                load_group(grp + 1)

            def step(i, carry, grp=grp, p=p):
                for b in range(2):
                    lc = i * 2 + b
                    wait_gather(b)
                    scale(p, lc, b)
                    scatter(p, lc, b)
                    if grp + 1 < ngrp:
                        @pl.when(lc + 2 < gch)
                        def _():
                            gather(p, lc + 2, b)

                        if b == 0:
                            @pl.when(lc + 2 == gch)
                            def _():
                                wait_group(grp + 1)
                                gather(1 - p, 0, b)
                        else:
                            @pl.when(lc + 2 == gch + 1)
                            def _():
                                gather(1 - p, 1, b)
                    else:
                        @pl.when(lc + 2 < gch)
                        def _():
                            gather(p, lc + 2, b)
                return carry

            lax.fori_loop(0, gch // 2, step, 0)

        plsc.subcore_barrier()
        pltpu.sync_copy(acc.at[pl.ds(r0, rpt)],
                        out_hbm.at[pl.ds(c * n_pad + r0, rpt)])

    out = k(row2d, col2d, val2d, h)
    return out[:n], out[n_pad:n_pad + n]


_BM = 512


def _tc_argmax(label_full):
    n, c = label_full.shape
    bm = 2000

    def body(lab_r, out_r):
        out_r[...] = jnp.argmax(lab_r[...], axis=1).astype(_I32)[:, None]

    return pl.pallas_call(
        body,
        grid=(pl.cdiv(n, bm),),
        in_specs=[pl.BlockSpec((bm, c), lambda i: (i, 0))],
        out_specs=pl.BlockSpec((bm, 1), lambda i: (i, 0)),
        out_shape=jax.ShapeDtypeStruct((n, 1), _I32),
    )(label_full)[:, 0]


def _tc_dense_a(feat, p0, p1, w00, b00, w01, b01, w10a, w10b, b10, w11a, w11b):
    n, d = feat.shape
    grid = (pl.cdiv(n, _BM),)
    row_spec = pl.BlockSpec((_BM, d), lambda i: (i, 0))
    w_spec = pl.BlockSpec((d, d), lambda i: (0, 0))
    b_spec = pl.BlockSpec((1, d), lambda i: (0, 0))

    def body(feat_r, p0_r, p1_r, w00r, b00r, w01r, b01r,
             w10ar, w10br, b10r, w11ar, w11br, g1_r, f0p_r):
        a0 = p0_r[...] + p1_r[...]
        f0 = jnp.maximum(
            jnp.dot(feat_r[...], w00r[...], preferred_element_type=_F32)
            + b00r[...], 0.0)
        f1 = jnp.maximum(
            jnp.dot(a0, w01r[...], preferred_element_type=_F32) + b01r[...],
            0.0)
        g1_r[...] = (jnp.dot(f0, w11ar[...], preferred_element_type=_F32)
                     + jnp.dot(f1, w11br[...], preferred_element_type=_F32))
        f0p_r[...] = jnp.maximum(
            jnp.dot(f0, w10ar[...], preferred_element_type=_F32)
            + jnp.dot(f1, w10br[...], preferred_element_type=_F32)
            + b10r[...], 0.0)

    return pl.pallas_call(
        body,
        grid=grid,
        in_specs=[row_spec, row_spec, row_spec,
                  w_spec, b_spec, w_spec, b_spec,
                  w_spec, w_spec, b_spec, w_spec, w_spec],
        out_specs=[row_spec, row_spec],
        out_shape=[jax.ShapeDtypeStruct((n, d), _F32),
                   jax.ShapeDtypeStruct((n, d), _F32)],
    )(feat, p0, p1, w00, b00, w01, b01, w10a, w10b, b10, w11a, w11b)


def _tc_dense_b(q0, q1, b11, f0p, wc_pad, bc_pad, conv2d, c_dim):
    n, d = f0p.shape
    dc = wc_pad.shape[1]
    grid = (pl.cdiv(n, _BM),)
    row_spec = pl.BlockSpec((_BM, d), lambda i: (i, 0))

    def body(q0_r, q1_r, b11r, f0p_r, wcr, bcr, conv_r, pred_r, lab_r):
        f1p = jnp.maximum(q0_r[...] + q1_r[...] + b11r[...], 0.0)
        h2 = jnp.concatenate([f0p_r[...], f1p], axis=1)
        nrm = jnp.sqrt(jnp.sum(h2 * h2, axis=1, keepdims=True))
        emb = h2 / jnp.maximum(nrm, 1e-12)
        pred_r[...] = (jnp.dot(emb, wcr[...], preferred_element_type=_F32)
                       + bcr[...])
        cols = lax.broadcasted_iota(_I32, (_BM, c_dim), 1)
        lab_r[...] = (cols == conv_r[...]).astype(_F32)

    return pl.pallas_call(
        body,
        grid=grid,
        in_specs=[row_spec, row_spec, pl.BlockSpec((1, d), lambda i: (0, 0)),
                  row_spec,
                  pl.BlockSpec((2 * d, dc), lambda i: (0, 0)),
                  pl.BlockSpec((1, dc), lambda i: (0, 0)),
                  pl.BlockSpec((_BM, 1), lambda i: (i, 0))],
        out_specs=[pl.BlockSpec((_BM, dc), lambda i: (i, 0)),
                   pl.BlockSpec((_BM, c_dim), lambda i: (i, 0))],
        out_shape=[jax.ShapeDtypeStruct((n, dc), _F32),
                   jax.ShapeDtypeStruct((n, c_dim), _F32)],
    )(q0, q1, b11, f0p, wc_pad, bc_pad, conv2d)


def kernel(node_subgraph, adj_row, adj_col, adj_val, feat_full, label_full,
           W0_0, b0_0, W0_1, b0_1, W1_0, b1_0, W1_1, b1_1, Wc, bc):
    n_sub = node_subgraph.shape[0]
    d = feat_full.shape[1]
    n_pad = ((n_sub + 255) // 256) * 256
    idx_pad = jnp.concatenate(
        [node_subgraph, jnp.zeros((n_pad - n_sub,), _I32)])

    e_sub = adj_row.shape[0]
    e_pad = ((e_sub + 16383) // 16384) * 16384
    epad = e_pad - e_sub
    row2d = jnp.concatenate([adj_row, jnp.zeros((epad,), _I32)]).reshape(-1, 128)
    col2d = jnp.concatenate([adj_col, jnp.zeros((epad,), _I32)]).reshape(-1, 128)
    val2d = jnp.concatenate([adj_val, jnp.zeros((epad,), _F32)]).reshape(-1, 128)

    lab_idx_full = _tc_argmax(label_full)
    feat_g, conv_g = _sc_gather(feat_full, lab_idx_full, idx_pad)
    feat_s = feat_g[:n_sub]
    conv = conv_g[:n_sub]

    p0, p1 = _sc_spmm(row2d, col2d, val2d, feat_s)

    g1, f0p = _tc_dense_a(
        feat_s, p0, p1,
        W0_0, b0_0[None], W0_1, b0_1[None],
        W1_0[:d], W1_0[d:], b1_0[None], W1_1[:d], W1_1[d:])

    q0, q1 = _sc_spmm(row2d, col2d, val2d, g1)

    dc = 128
    c_dim = Wc.shape[1]
    wc_pad = jnp.pad(Wc, ((0, 0), (0, dc - c_dim)))
    bc_pad = jnp.pad(bc, (0, dc - bc.shape[0]))[None]
    pred_pad, lab_s = _tc_dense_b(
        q0, q1, b1_1[None], f0p, wc_pad, bc_pad,
        conv[:, None], c_dim)

    return pred_pad[:, :c_dim], lab_s, conv


# speedup vs baseline: 6.0747x; 1.6654x over previous
import functools

import jax
import jax.numpy as jnp
from jax import lax
from jax.experimental import pallas as pl
from jax.experimental.pallas import tpu as pltpu
from jax.experimental.pallas import tpu_sc as plsc

_F32 = jnp.float32
_I32 = jnp.int32


def _sc_gather(feat_full, lab_idx_full, idx_pad):
    n_full, d = feat_full.shape
    b = idx_pad.shape[0]
    bpw = b // 32

    mesh = plsc.VectorSubcoreMesh(core_axis_name="c", subcore_axis_name="s")

    @functools.partial(
        pl.kernel,
        out_type=(
            jax.ShapeDtypeStruct((b, d), _F32),
            jax.ShapeDtypeStruct((b,), _I32),
        ),
        mesh=mesh,
        compiler_params=pltpu.CompilerParams(needs_layout_passes=False),
        scratch_types=[
            pltpu.VMEM((bpw // 2,), _I32),
            pltpu.VMEM((bpw // 2,), _I32),
            pltpu.VMEM((bpw // 2, d), _F32),
            pltpu.VMEM((n_full,), _I32),
            pltpu.VMEM((bpw,), _I32),
            pltpu.SemaphoreType.DMA,
            pltpu.SemaphoreType.DMA,
        ],
    )
    def k(feat_hbm, lab_hbm, idx_hbm, feat_out, lab_out,
          idx_a, idx_b, frows, lab_v, lout_v, sem, sem2):
        c = lax.axis_index("c")
        s = lax.axis_index("s")
        wid = s * 2 + c
        base = wid * bpw
        half = bpw // 2
        pltpu.sync_copy(idx_hbm.at[pl.ds(base, half)], idx_a)
        pltpu.sync_copy(idx_hbm.at[pl.ds(base + half, half)], idx_b)
        cp = pltpu.make_async_copy(lab_hbm, lab_v, sem2)
        cp.start()
        pltpu.async_copy(feat_hbm.at[idx_a], frows, sem).wait()
        pltpu.sync_copy(frows, feat_out.at[pl.ds(base, half)])
        pltpu.async_copy(feat_hbm.at[idx_b], frows, sem).wait()
        pltpu.sync_copy(frows, feat_out.at[pl.ds(base + half, half)])
        cp.wait()
        for j in range(half // 16):
            iv = idx_a[pl.ds(j * 16, 16)]
            lout_v[pl.ds(j * 16, 16)] = plsc.load_gather(lab_v, [iv])
        for j in range(half // 16):
            iv = idx_b[pl.ds(j * 16, 16)]
            lout_v[pl.ds(half + j * 16, 16)] = plsc.load_gather(lab_v, [iv])
        pltpu.sync_copy(lout_v, lab_out.at[pl.ds(base, bpw)])

    return k(feat_full, lab_idx_full, idx_pad)


def _sc_spmm(row, col, val, h):
    e_total = row.shape[0]
    n, d = h.shape
    n_pad = ((n + 511) // 512) * 512
    epw = e_total // 32
    k_ch = 80
    n_ch = epw // k_ch
    rpt = n_pad // 16
    zr = 32
    ng = d // 16

    mesh = plsc.VectorSubcoreMesh(core_axis_name="c", subcore_axis_name="s")

    @functools.partial(
        pl.kernel,
        out_type=jax.ShapeDtypeStruct((2 * n_pad, d), _F32),
        mesh=mesh,
        compiler_params=pltpu.CompilerParams(needs_layout_passes=False),
        scratch_types=[
            [pltpu.VMEM((k_ch,), _I32)] * 2,
            [pltpu.VMEM((k_ch,), _I32)] * 2,
            [pltpu.VMEM((k_ch,), _F32)] * 2,
            [pltpu.VMEM((k_ch, d), _F32)] * 2,
            pltpu.VMEM((zr, d), _F32),
            pltpu.MemorySpace.VMEM_SHARED((n_pad, d), _F32),
            [pltpu.SemaphoreType.DMA] * 2,
        ],
    )
    def k(row_hbm, col_hbm, val_hbm, h_hbm, out_hbm,
          colv, rowv, valv, rows, zbuf, acc, gsem):
        c = lax.axis_index("c")
        s = lax.axis_index("s")
        wid = s * 2 + c
        e_base = wid * epw

        def load_idx(ch, b):
            e0 = e_base + ch * k_ch
            pltpu.sync_copy(col_hbm.at[pl.ds(e0, k_ch)], colv[b])
            pltpu.sync_copy(row_hbm.at[pl.ds(e0, k_ch)], rowv[b])
            pltpu.sync_copy(val_hbm.at[pl.ds(e0, k_ch)], valv[b])

        def gather(b):
            pltpu.async_copy(h_hbm.at[colv[b]], rows[b], gsem[b])

        def wait_gather(b):
            pltpu.make_async_copy(h_hbm.at[colv[b]], rows[b], gsem[b]).wait()

        def scale(b):
            def sj(j):
                vv = valv[b][pl.ds(j * 16, 16)]
                for t in range(16):
                    for g in range(ng):
                        sl = pl.ds(g * 16, 16)
                        rows[b][j * 16 + t, sl] = rows[b][j * 16 + t, sl] * vv[t]
            plsc.parallel_loop(0, k_ch // 16)(sj)

        zero16 = jnp.zeros((16,), _F32)
        for i in range(zr):
            for g in range(ng):
                zbuf[i, pl.ds(g * 16, 16)] = zero16
        r0 = s * rpt
        for j in range(rpt // zr):
            pltpu.sync_copy(zbuf, acc.at[pl.ds(r0 + j * zr, zr)])
        plsc.subcore_barrier()

        load_idx(0, 0)
        gather(0)
        load_idx(1, 1)
        gather(1)

        def step(i, carry):
            for b in range(2):
                ch = i * 2 + b
                wait_gather(b)
                scale(b)
                pltpu.sync_copy(rows[b], acc.at[rowv[b]], add=True)

                @pl.when(ch + 2 < n_ch)
                def _():
                    load_idx(ch + 2, b)
                    gather(b)
            return carry

        lax.fori_loop(0, n_ch // 2, step, 0)
        plsc.subcore_barrier()
        pltpu.sync_copy(acc.at[pl.ds(r0, rpt)],
                        out_hbm.at[pl.ds(c * n_pad + r0, rpt)])

    out = k(row, col, val, h)
    return out[:n], out[n_pad:n_pad + n]


_BM = 512


def _tc_argmax(label_full):
    n, c = label_full.shape
    bm = 2000

    def body(lab_r, out_r):
        out_r[...] = jnp.argmax(lab_r[...], axis=1).astype(_I32)[:, None]

    return pl.pallas_call(
        body,
        grid=(pl.cdiv(n, bm),),
        in_specs=[pl.BlockSpec((bm, c), lambda i: (i, 0))],
        out_specs=pl.BlockSpec((bm, 1), lambda i: (i, 0)),
        out_shape=jax.ShapeDtypeStruct((n, 1), _I32),
    )(label_full)[:, 0]


def _tc_dense_a(feat, p0, p1, w00, b00, w01, b01, w10a, w10b, b10, w11a, w11b):
    n, d = feat.shape
    grid = (pl.cdiv(n, _BM),)
    row_spec = pl.BlockSpec((_BM, d), lambda i: (i, 0))
    w_spec = pl.BlockSpec((d, d), lambda i: (0, 0))
    b_spec = pl.BlockSpec((1, d), lambda i: (0, 0))

    def body(feat_r, p0_r, p1_r, w00r, b00r, w01r, b01r,
             w10ar, w10br, b10r, w11ar, w11br, g1_r, f0p_r):
        a0 = p0_r[...] + p1_r[...]
        f0 = jnp.maximum(
            jnp.dot(feat_r[...], w00r[...], preferred_element_type=_F32)
            + b00r[...], 0.0)
        f1 = jnp.maximum(
            jnp.dot(a0, w01r[...], preferred_element_type=_F32) + b01r[...],
            0.0)
        g1_r[...] = (jnp.dot(f0, w11ar[...], preferred_element_type=_F32)
                     + jnp.dot(f1, w11br[...], preferred_element_type=_F32))
        f0p_r[...] = jnp.maximum(
            jnp.dot(f0, w10ar[...], preferred_element_type=_F32)
            + jnp.dot(f1, w10br[...], preferred_element_type=_F32)
            + b10r[...], 0.0)

    return pl.pallas_call(
        body,
        grid=grid,
        in_specs=[row_spec, row_spec, row_spec,
                  w_spec, b_spec, w_spec, b_spec,
                  w_spec, w_spec, b_spec, w_spec, w_spec],
        out_specs=[row_spec, row_spec],
        out_shape=[jax.ShapeDtypeStruct((n, d), _F32),
                   jax.ShapeDtypeStruct((n, d), _F32)],
    )(feat, p0, p1, w00, b00, w01, b01, w10a, w10b, b10, w11a, w11b)


def _tc_dense_b(q0, q1, b11, f0p, wc_pad, bc_pad, conv2d, c_dim):
    n, d = f0p.shape
    dc = wc_pad.shape[1]
    grid = (pl.cdiv(n, _BM),)
    row_spec = pl.BlockSpec((_BM, d), lambda i: (i, 0))

    def body(q0_r, q1_r, b11r, f0p_r, wcr, bcr, conv_r, pred_r, lab_r):
        f1p = jnp.maximum(q0_r[...] + q1_r[...] + b11r[...], 0.0)
        h2 = jnp.concatenate([f0p_r[...], f1p], axis=1)
        nrm = jnp.sqrt(jnp.sum(h2 * h2, axis=1, keepdims=True))
        emb = h2 / jnp.maximum(nrm, 1e-12)
        pred_r[...] = (jnp.dot(emb, wcr[...], preferred_element_type=_F32)
                       + bcr[...])
        cols = lax.broadcasted_iota(_I32, (_BM, c_dim), 1)
        lab_r[...] = (cols == conv_r[...]).astype(_F32)

    return pl.pallas_call(
        body,
        grid=grid,
        in_specs=[row_spec, row_spec, pl.BlockSpec((1, d), lambda i: (0, 0)),
                  row_spec,
                  pl.BlockSpec((2 * d, dc), lambda i: (0, 0)),
                  pl.BlockSpec((1, dc), lambda i: (0, 0)),
                  pl.BlockSpec((_BM, 1), lambda i: (i, 0))],
        out_specs=[pl.BlockSpec((_BM, dc), lambda i: (i, 0)),
                   pl.BlockSpec((_BM, c_dim), lambda i: (i, 0))],
        out_shape=[jax.ShapeDtypeStruct((n, dc), _F32),
                   jax.ShapeDtypeStruct((n, c_dim), _F32)],
    )(q0, q1, b11, f0p, wc_pad, bc_pad, conv2d)


def kernel(node_subgraph, adj_row, adj_col, adj_val, feat_full, label_full,
           W0_0, b0_0, W0_1, b0_1, W1_0, b1_0, W1_1, b1_1, Wc, bc):
    n_sub = node_subgraph.shape[0]
    d = feat_full.shape[1]
    n_pad = ((n_sub + 255) // 256) * 256
    idx_pad = jnp.concatenate(
        [node_subgraph, jnp.zeros((n_pad - n_sub,), _I32)])

    lab_idx_full = _tc_argmax(label_full)
    feat_g, conv_g = _sc_gather(feat_full, lab_idx_full, idx_pad)
    feat_s = feat_g[:n_sub]
    conv = conv_g[:n_sub]

    p0, p1 = _sc_spmm(adj_row, adj_col, adj_val, feat_s)

    g1, f0p = _tc_dense_a(
        feat_s, p0, p1,
        W0_0, b0_0[None], W0_1, b0_1[None],
        W1_0[:d], W1_0[d:], b1_0[None], W1_1[:d], W1_1[d:])

    q0, q1 = _sc_spmm(adj_row, adj_col, adj_val, g1)

    dc = 128
    c_dim = Wc.shape[1]
    wc_pad = jnp.pad(Wc, ((0, 0), (0, dc - c_dim)))
    bc_pad = jnp.pad(bc, (0, dc - bc.shape[0]))[None]
    pred_pad, lab_s = _tc_dense_b(
        q0, q1, b1_1[None], f0p, wc_pad, bc_pad,
        conv[:, None], c_dim)

    return pred_pad[:, :c_dim], lab_s, conv
